# Initial kernel scaffold; baseline (speedup 1.0000x reference)
import jax
import jax.numpy as jnp
from jax.experimental import pallas as pl


def kernel(x, edge_index, edge_attr, e1_fcx_W, e1_fcx_b, e1_fce_W, e1_fce_b, n1_fcx_W, n1_fcx_b, n1_fce_W, n1_fce_b):
    raise NotImplementedError("write your pallas kernel here")


# speedup vs baseline: 37.2061x; 37.2061x over previous
import jax
import jax.numpy as jnp
from jax import lax
from jax.experimental import pallas as pl
from jax.experimental.pallas import tpu as pltpu
from jax.experimental.pallas import tpu_sc as plsc

N_NODES = 100000
N_EDGES = 6400000

NC = 2
NS = 16
NW = NC * NS
L = 16

EPT = N_EDGES // NW
CHUNK = 1600
NCHUNK = EPT // CHUNK
NSTEP = CHUNK // L

N_PAD = 100352
RPT = N_PAD // NS
NPT = N_PAD // NW

_MASK_HI = -65536


def _edge_body(xp_hbm, src_hbm, dst_hbm, eattr_hbm, eparams_hbm, *refs):
    ea_hbms = refs[:6]
    xp_v, src_v, dst_v, attr_v, eac_v, prm_v = refs[6:]
    c = lax.axis_index("c")
    s = lax.axis_index("s")
    wid = c * NS + s

    pltpu.sync_copy(xp_hbm, xp_v)
    pltpu.sync_copy(eparams_hbm, prm_v)

    pv = prm_v[pl.ds(0, 16)]
    we0 = pv[0]
    we1 = pv[1]
    be0 = pv[2]
    be1 = pv[3]
    wx00 = pv[4]
    wx01 = pv[5]
    wx10 = pv[6]
    wx11 = pv[7]
    wx20 = pv[8]
    wx21 = pv[9]
    wx30 = pv[10]
    wx31 = pv[11]
    bx0 = pv[12]
    bx1 = pv[13]
    bx2 = pv[14]
    bx3 = pv[15]
    zero = jnp.float32(0.0)

    def _step(j, _):
        sl = pl.ds(j * L, L)
        si = src_v[sl]
        di = dst_v[sl]
        at = attr_v[sl]
        ps = plsc.load_gather(xp_v, [si])
        pd = plsc.load_gather(xp_v, [di])
        x0 = plsc.bitcast(lax.shift_left(ps, 16), jnp.float32) + \
            plsc.bitcast(lax.shift_left(pd, 16), jnp.float32)
        x1 = plsc.bitcast(lax.bitwise_and(ps, _MASK_HI), jnp.float32) + \
            plsc.bitcast(lax.bitwise_and(pd, _MASK_HI), jnp.float32)
        eac_v[0, sl] = jnp.maximum(at * we0 + be0, zero)
        eac_v[1, sl] = jnp.maximum(at * we1 + be1, zero)
        eac_v[2, sl] = jnp.maximum(x0 * wx00 + x1 * wx01 + bx0, zero)
        eac_v[3, sl] = jnp.maximum(x0 * wx10 + x1 * wx11 + bx1, zero)
        eac_v[4, sl] = jnp.maximum(x0 * wx20 + x1 * wx21 + bx2, zero)
        eac_v[5, sl] = jnp.maximum(x0 * wx30 + x1 * wx31 + bx3, zero)
        return 0

    def _chunk(i, _):
        base = wid * EPT + i * CHUNK
        pltpu.sync_copy(src_hbm.at[pl.ds(base, CHUNK)], src_v)
        pltpu.sync_copy(dst_hbm.at[pl.ds(base, CHUNK)], dst_v)
        pltpu.sync_copy(eattr_hbm.at[pl.ds(base, CHUNK)], attr_v)
        lax.fori_loop(0, NSTEP, _step, 0)
        dsl = pl.ds(base, CHUNK)
        for cc in range(6):
            pltpu.sync_copy(eac_v.at[cc], ea_hbms[cc].at[dsl])
        return 0

    lax.fori_loop(0, NCHUNK, _chunk, 0)


def _scatter_body(src_hbm, dst_hbm, ea_hbms, part_hbm,
                  src_v, dst_v, eac_v, ea_v, esum):
    c = lax.axis_index("c")
    s = lax.axis_index("s")
    wid = c * NS + s

    iota = lax.iota(jnp.int32, L)
    zvec = jnp.zeros((L,), jnp.float32)
    cols = [jnp.full((L,), cc, jnp.int32) for cc in range(8)]

    def _zbuf(j, _):
        rows = iota + j * L
        for cc in range(8):
            plsc.store_scatter(ea_v, [rows, cols[cc]], zvec)
        return 0

    lax.fori_loop(0, NSTEP, _zbuf, 0)

    def _zero(i, _):
        pltpu.sync_copy(ea_v, esum.at[pl.ds(s * RPT + i * CHUNK, CHUNK)])
        return 0

    nz = RPT // CHUNK
    lax.fori_loop(0, nz, _zero, 0)
    rem = RPT - nz * CHUNK
    pltpu.sync_copy(ea_v.at[pl.ds(0, rem)],
                    esum.at[pl.ds(s * RPT + nz * CHUNK, rem)])
    plsc.subcore_barrier()

    def _step(j, _):
        sl = pl.ds(j * L, L)
        rows = iota + j * L
        for cc in range(6):
            plsc.store_scatter(ea_v, [rows, cols[cc]], eac_v[cc, sl])
        return 0

    def _chunk(i, _):
        base = wid * EPT + i * CHUNK
        pltpu.sync_copy(src_hbm.at[pl.ds(base, CHUNK)], src_v)
        pltpu.sync_copy(dst_hbm.at[pl.ds(base, CHUNK)], dst_v)
        dsl = pl.ds(base, CHUNK)
        for cc in range(6):
            pltpu.sync_copy(ea_hbms[cc].at[dsl], eac_v.at[cc])
        lax.fori_loop(0, NSTEP, _step, 0)
        pltpu.sync_copy(ea_v, esum.at[dst_v], add=True)
        pltpu.sync_copy(ea_v, esum.at[src_v], add=True)
        return 0

    lax.fori_loop(0, NCHUNK, _chunk, 0)
    plsc.subcore_barrier()

    pltpu.sync_copy(esum.at[pl.ds(s * RPT, RPT)],
                    part_hbm.at[pl.ds(c * N_PAD + s * RPT, RPT)])


def _node_body(xp_hbm, part_hbm, nparams_hbm, *refs):
    xn_hbms = refs[:14]
    xq_v, p0_v, p1_v, xnc_v, prm_v = refs[14:]
    c = lax.axis_index("c")
    s = lax.axis_index("s")
    wid = c * NS + s
    base = wid * NPT

    pltpu.sync_copy(nparams_hbm, prm_v)
    pltpu.sync_copy(xp_hbm.at[pl.ds(base, NPT)], xq_v)
    pltpu.sync_copy(part_hbm.at[pl.ds(base, NPT)], p0_v)
    pltpu.sync_copy(part_hbm.at[pl.ds(N_PAD + base, NPT)], p1_v)

    pvs = [prm_v[pl.ds(16 * k, 16)] for k in range(6)]
    pva = pvs[0]

    def _w2(r, k):
        i = 12 + r * 6 + k
        return pvs[i // 16][i % 16]

    def _b2(r):
        i = 72 + r
        return pvs[i // 16][i % 16]

    iota = lax.iota(jnp.int32, L)
    zero = jnp.float32(0.0)
    col6 = [jnp.full((L,), cc, jnp.int32) for cc in range(6)]

    def _step(j, _):
        rows = iota + j * L
        sl = pl.ds(j * L, L)
        q = xq_v[sl]
        x0 = plsc.bitcast(lax.shift_left(q, 16), jnp.float32)
        x1 = plsc.bitcast(lax.bitwise_and(q, _MASK_HI), jnp.float32)
        es = [plsc.load_gather(p0_v, [rows, col6[k]]) +
              plsc.load_gather(p1_v, [rows, col6[k]]) for k in range(6)]
        for r in range(4):
            h = jnp.maximum(x0 * pva[r * 2] + x1 * pva[r * 2 + 1] + pva[8 + r],
                            zero)
            xnc_v[r, sl] = h
        for r in range(10):
            acc = es[0] * _w2(r, 0)
            for k in range(1, 6):
                acc = acc + es[k] * _w2(r, k)
            h = jnp.maximum(acc + _b2(r), zero)
            xnc_v[4 + r, sl] = h
        return 0

    lax.fori_loop(0, NPT // L, _step, 0)
    dsl = pl.ds(base, NPT)
    for r in range(14):
        pltpu.sync_copy(xnc_v.at[r], xn_hbms[r].at[dsl])


def kernel(x, edge_index, edge_attr,
           e1_fcx_W, e1_fcx_b, e1_fce_W, e1_fce_b,
           n1_fcx_W, n1_fcx_b, n1_fce_W, n1_fce_b):
    xu = lax.bitcast_convert_type(x.astype(jnp.bfloat16), jnp.uint16)
    xp = lax.bitcast_convert_type(
        xu[:, 0].astype(jnp.uint32) | (xu[:, 1].astype(jnp.uint32) << 16),
        jnp.int32)
    xp = jnp.concatenate([xp, jnp.zeros((N_PAD - N_NODES,), jnp.int32)])
    srcs = edge_index[0]
    dsts = edge_index[1]
    eattr = edge_attr.reshape(N_EDGES)
    eparams = jnp.concatenate([
        e1_fce_W.reshape(2), e1_fce_b,
        e1_fcx_W.reshape(8), e1_fcx_b]).astype(jnp.float32)
    nparams = jnp.concatenate([
        n1_fcx_W.reshape(8), n1_fcx_b,
        n1_fce_W.reshape(60), n1_fce_b,
        jnp.zeros((14,), jnp.float32)]).astype(jnp.float32)

    mesh = plsc.VectorSubcoreMesh(core_axis_name="c", subcore_axis_name="s",
                                  num_cores=NC, num_subcores=NS)
    params = pltpu.CompilerParams(needs_layout_passes=False,
                                  use_tc_tiling_on_sc=False)

    edge_k = pl.kernel(
        _edge_body,
        out_type=tuple(jax.ShapeDtypeStruct((N_EDGES,), jnp.float32)
                       for _ in range(6)),
        mesh=mesh,
        compiler_params=params,
        scratch_types=[
            pltpu.VMEM((N_PAD,), jnp.int32),
            pltpu.VMEM((CHUNK,), jnp.int32),
            pltpu.VMEM((CHUNK,), jnp.int32),
            pltpu.VMEM((CHUNK,), jnp.float32),
            pltpu.VMEM((6, CHUNK), jnp.float32),
            pltpu.VMEM((16,), jnp.float32),
        ],
    )
    ea_cols = edge_k(xp, srcs, dsts, eattr, eparams)

    scatter_k = pl.kernel(
        _scatter_body,
        out_type=jax.ShapeDtypeStruct((NC * N_PAD, 8), jnp.float32),
        mesh=mesh,
        compiler_params=params,
        scratch_types=[
            pltpu.VMEM((CHUNK,), jnp.int32),
            pltpu.VMEM((CHUNK,), jnp.int32),
            pltpu.VMEM((6, CHUNK), jnp.float32),
            pltpu.VMEM((CHUNK, 8), jnp.float32),
            pltpu.VMEM_SHARED((N_PAD, 8), jnp.float32),
        ],
    )
    part = scatter_k(srcs, dsts, ea_cols)

    node_k = pl.kernel(
        _node_body,
        out_type=tuple(jax.ShapeDtypeStruct((N_PAD,), jnp.float32)
                       for _ in range(14)),
        mesh=mesh,
        compiler_params=params,
        scratch_types=[
            pltpu.VMEM((NPT,), jnp.int32),
            pltpu.VMEM((NPT, 8), jnp.float32),
            pltpu.VMEM((NPT, 8), jnp.float32),
            pltpu.VMEM((14, NPT), jnp.float32),
            pltpu.VMEM((96,), jnp.float32),
        ],
    )
    xn_cols = node_k(xp, part, nparams)
    xn = jnp.stack([col[:N_NODES] for col in xn_cols], axis=1)
    ea = jnp.stack(list(ea_cols), axis=1)
    return xn, ea
